# Initial kernel scaffold; baseline (speedup 1.0000x reference)
import functools

import jax
import jax.numpy as jnp
from jax import lax
from jax.experimental import pallas as pl
from jax.experimental.pallas import tpu as pltpu
from jax.experimental.pallas import tpu_sc as plsc

N = 10000
E = 320000
NC, NS, L = 2, 16, 16
NW = NC * NS
EPW = E // NW
BLK = 100
NBLK = EPW // BLK
NPAD = 10240
RPS = N // NS
ZR = 125

_mesh = plsc.VectorSubcoreMesh(core_axis_name="c", subcore_axis_name="s")


@functools.partial(
    pl.kernel, mesh=_mesh,
    out_type=jax.ShapeDtypeStruct((NW, NPAD), jnp.float32),
    scratch_types=[
        pltpu.VMEM((EPW // L, L), jnp.int32),
        pltpu.VMEM((NPAD,), jnp.float32),
        pltpu.SemaphoreType.DMA,
    ],
)
def _deg_kernel(dst_hbm, out_hbm, dst_v, hist, sem):
    wid = lax.axis_index("s") * NC + lax.axis_index("c")
    pltpu.async_copy(dst_hbm.at[wid], dst_v, sem).wait()

    zero16 = jnp.zeros((L,), jnp.float32)

    @pl.loop(0, NPAD // L)
    def _(i):
        hist[pl.ds(i * L, L)] = zero16

    ones16 = jnp.ones((L,), jnp.float32)

    @pl.loop(0, EPW // L)
    def _(j):
        plsc.addupdate_scatter(hist, [dst_v[j, :]], ones16)

    pltpu.async_copy(hist, out_hbm.at[wid], sem).wait()


def _make_agg(F):
    @functools.partial(
        pl.kernel, mesh=_mesh,
        out_type=jax.ShapeDtypeStruct((NC, N, F), jnp.float32),
        scratch_types=[
            pltpu.VMEM((NBLK, BLK), jnp.int32),
            pltpu.VMEM((NBLK, BLK), jnp.int32),
            pltpu.VMEM((BLK, F), jnp.float32),
            pltpu.VMEM((ZR, F), jnp.float32),
            pltpu.VMEM_SHARED((N, F), jnp.float32),
            pltpu.SemaphoreType.DMA,
            pltpu.SemaphoreType.DMA,
        ],
    )
    def agg(table_hbm, src_hbm, dst_hbm, out_hbm, src_v, dst_v, buf, zbuf,
            acc, sem, semi):
        c = lax.axis_index("c")
        s = lax.axis_index("s")
        wid = s * NC + c
        pltpu.async_copy(src_hbm.at[wid], src_v, semi).wait()
        pltpu.async_copy(dst_hbm.at[wid], dst_v, semi).wait()

        zero16 = jnp.zeros((L,), jnp.float32)

        @pl.loop(0, ZR)
        def _(r):
            @pl.loop(0, F // L)
            def _(k):
                zbuf[r, pl.ds(k * L, L)] = zero16

        @pl.loop(0, RPS // ZR)
        def _(t):
            pltpu.sync_copy(zbuf, acc.at[pl.ds(s * RPS + t * ZR, ZR)])

        plsc.subcore_barrier()

        @pl.loop(0, NBLK)
        def _(j):
            pltpu.async_copy(table_hbm.at[src_v.at[j]], buf, sem).wait()
            pltpu.sync_copy(buf, acc.at[dst_v.at[j]], add=True)

        plsc.subcore_barrier()
        pltpu.async_copy(acc.at[pl.ds(s * RPS, RPS)],
                         out_hbm.at[c, pl.ds(s * RPS, RPS)], sem).wait()

    return agg


_agg64 = _make_agg(64)
_agg32 = _make_agg(32)


def _mm_body(x_ref, w_ref, o_ref):
    o_ref[...] = jnp.dot(x_ref[...], w_ref[...],
                         preferred_element_type=jnp.float32)


def _mm_xw(x, W):
    k, f = W.shape
    return pl.pallas_call(
        _mm_body,
        grid=(5,),
        in_specs=[pl.BlockSpec((2000, k), lambda i: (i, 0)),
                  pl.BlockSpec((k, f), lambda i: (0, 0))],
        out_specs=pl.BlockSpec((2000, f), lambda i: (i, 0)),
        out_shape=jax.ShapeDtypeStruct((N, f), jnp.float32),
    )(x, W)


def _dinv_body(p_ref, o_ref):
    s = jnp.sum(p_ref[...], axis=0, keepdims=True) + 1.0
    o_ref[...] = lax.rsqrt(s)


def _dinv_row(parts):
    return pl.pallas_call(
        _dinv_body,
        in_specs=[pl.BlockSpec((NW, NPAD), lambda: (0, 0))],
        out_specs=pl.BlockSpec((1, NPAD), lambda: (0, 0)),
        out_shape=jax.ShapeDtypeStruct((1, NPAD), jnp.float32),
    )(parts)


def _scale_body(a_ref, d_ref, o_ref):
    o_ref[...] = a_ref[...] * d_ref[...]


def _scale_rows(a, dcol):
    f = a.shape[1]
    return pl.pallas_call(
        _scale_body,
        grid=(5,),
        in_specs=[pl.BlockSpec((2000, f), lambda i: (i, 0)),
                  pl.BlockSpec((2000, 1), lambda i: (i, 0))],
        out_specs=pl.BlockSpec((2000, f), lambda i: (i, 0)),
        out_shape=jax.ShapeDtypeStruct((N, f), jnp.float32),
    )(a, dcol)


def _mid_body(p_ref, xs_ref, d_ref, b1_ref, w2_ref, o_ref):
    e = p_ref[0] + p_ref[1] + xs_ref[...]
    h = jnp.maximum(d_ref[...] * e + b1_ref[...], 0.0)
    o_ref[...] = d_ref[...] * jnp.dot(h, w2_ref[...],
                                      preferred_element_type=jnp.float32)


def _mid(p, xs1, dcol, b1, W2):
    return pl.pallas_call(
        _mid_body,
        grid=(5,),
        in_specs=[pl.BlockSpec((NC, 2000, 64), lambda i: (0, i, 0)),
                  pl.BlockSpec((2000, 64), lambda i: (i, 0)),
                  pl.BlockSpec((2000, 1), lambda i: (i, 0)),
                  pl.BlockSpec((1, 64), lambda i: (0, 0)),
                  pl.BlockSpec((64, 32), lambda i: (0, 0))],
        out_specs=pl.BlockSpec((2000, 32), lambda i: (i, 0)),
        out_shape=jax.ShapeDtypeStruct((N, 32), jnp.float32),
    )(p, xs1, dcol, b1, W2)


def _dec_body(q_ref, hs_ref, d_ref, b2_ref, wf1_ref, bf1_ref, wf2_ref,
              bf2_ref, o_ref):
    z = d_ref[...] * (q_ref[0] + q_ref[1] + hs_ref[...]) + b2_ref[...]
    dd = jnp.maximum(jnp.dot(z, wf1_ref[...],
                             preferred_element_type=jnp.float32)
                     + bf1_ref[...], 0.0)
    o_ref[...] = jnp.maximum(jnp.dot(dd, wf2_ref[...],
                                     preferred_element_type=jnp.float32)
                             + bf2_ref[...], 0.0)


def _decode(q, hs, dcol, b2, Wf1, bf1, Wf2, bf2):
    return pl.pallas_call(
        _dec_body,
        grid=(5,),
        in_specs=[pl.BlockSpec((NC, 2000, 32), lambda i: (0, i, 0)),
                  pl.BlockSpec((2000, 32), lambda i: (i, 0)),
                  pl.BlockSpec((2000, 1), lambda i: (i, 0)),
                  pl.BlockSpec((1, 32), lambda i: (0, 0)),
                  pl.BlockSpec((32, 64), lambda i: (0, 0)),
                  pl.BlockSpec((1, 64), lambda i: (0, 0)),
                  pl.BlockSpec((64, 128), lambda i: (0, 0)),
                  pl.BlockSpec((1, 128), lambda i: (0, 0))],
        out_specs=pl.BlockSpec((2000, 128), lambda i: (i, 0)),
        out_shape=jax.ShapeDtypeStruct((N, 128), jnp.float32),
    )(q, hs, dcol, b2, Wf1, bf1, Wf2, bf2)


def kernel(x, edge_index, W1, b1, W2, b2, Wf1, bf1, Wf2, bf2):
    ei = edge_index.astype(jnp.int32)
    src3 = ei[0].reshape(NW, NBLK, BLK)
    dst3 = ei[1].reshape(NW, NBLK, BLK)
    dst_deg = ei[1].reshape(NW, EPW // L, L)

    deg_parts = _deg_kernel(dst_deg)
    xw1 = _mm_xw(x, W1)

    dinv = _dinv_row(deg_parts)
    dcol = dinv[0, :N].reshape(N, 1)

    xs1 = _scale_rows(xw1, dcol)
    p = _agg64(xs1, src3, dst3)
    hs = _mid(p, xs1, dcol, b1.reshape(1, 64), W2)
    q = _agg32(hs, src3, dst3)
    return _decode(q, hs, dcol, b2.reshape(1, 32), Wf1, bf1.reshape(1, 64),
                   Wf2, bf2.reshape(1, 128))


# speedup vs baseline: 28.8081x; 28.8081x over previous
import dataclasses
import functools

import jax
import jax.numpy as jnp
from jax import lax
from jax.experimental import pallas as pl
from jax.experimental.pallas import tpu as pltpu
from jax.experimental.pallas import tpu_sc as plsc

N = 10000
E = 320000
NC, NS, L = 2, 16, 16
NW = NC * NS
EPW = E // NW
BLK = 100
NBLK = EPW // BLK
NPAD = 10240
ZR = 128
CP0 = 624
CP1 = N - 15 * CP0

_mesh = plsc.VectorSubcoreMesh(core_axis_name="c", subcore_axis_name="s")

_sc_params = pltpu.CompilerParams()
for _f, _v in (("needs_layout_passes", False), ("use_tc_tiling_on_sc", False)):
    if _f in pltpu.CompilerParams.__dataclass_fields__:
        _sc_params = dataclasses.replace(_sc_params, **{_f: _v})


@functools.partial(
    pl.kernel, mesh=_mesh, compiler_params=_sc_params,
    out_type=jax.ShapeDtypeStruct((NW, NPAD), jnp.float32),
    scratch_types=[
        pltpu.VMEM((EPW // L, L), jnp.int32),
        pltpu.VMEM((NPAD,), jnp.float32),
        pltpu.SemaphoreType.DMA,
    ],
)
def _deg_kernel(dst_hbm, out_hbm, dst_v, hist, sem):
    wid = lax.axis_index("s") * NC + lax.axis_index("c")
    pltpu.async_copy(dst_hbm.at[wid], dst_v, sem).wait()

    zero16 = jnp.zeros((L,), jnp.float32)

    @pl.loop(0, NPAD // L)
    def _(i):
        hist[pl.ds(i * L, L)] = zero16

    ones16 = jnp.ones((L,), jnp.float32)

    @pl.loop(0, EPW // L)
    def _(j):
        plsc.addupdate_scatter(hist, [dst_v[j, :]], ones16)

    pltpu.async_copy(hist, out_hbm.at[wid], sem).wait()


def _make_agg(F):
    @functools.partial(
        pl.kernel, mesh=_mesh, compiler_params=_sc_params,
        out_type=jax.ShapeDtypeStruct((NC, N, F), jnp.float32),
        scratch_types=[
            pltpu.VMEM((NBLK, BLK), jnp.int32),
            pltpu.VMEM((NBLK, BLK), jnp.int32),
            pltpu.VMEM((BLK, F), jnp.float32),
            pltpu.VMEM((ZR, F), jnp.float32),
            pltpu.VMEM_SHARED((NPAD, F), jnp.float32),
            pltpu.SemaphoreType.DMA,
            pltpu.SemaphoreType.DMA,
        ],
    )
    def agg(table_hbm, src_hbm, dst_hbm, out_hbm, src_v, dst_v, buf, zbuf,
            acc, sem, semi):
        c = lax.axis_index("c")
        s = lax.axis_index("s")
        wid = s * NC + c
        pltpu.async_copy(src_hbm.at[wid], src_v, semi).wait()
        pltpu.async_copy(dst_hbm.at[wid], dst_v, semi).wait()

        zero16 = jnp.zeros((L,), jnp.float32)

        @pl.loop(0, ZR)
        def _(r):
            @pl.loop(0, F // L)
            def _(k):
                zbuf[r, pl.ds(k * L, L)] = zero16

        @pl.loop(0, NPAD // NS // ZR)
        def _(t):
            pltpu.sync_copy(zbuf, acc.at[pl.ds(s * (NPAD // NS) + t * ZR, ZR)])

        plsc.subcore_barrier()

        @pl.loop(0, NBLK)
        def _(j):
            pltpu.async_copy(table_hbm.at[src_v.at[j]], buf, sem).wait()
            pltpu.sync_copy(buf, acc.at[dst_v.at[j]], add=True)

        plsc.subcore_barrier()

        @pl.when(s < NS - 1)
        def _():
            pltpu.async_copy(acc.at[pl.ds(s * CP0, CP0)],
                             out_hbm.at[c, pl.ds(s * CP0, CP0)], sem).wait()

        @pl.when(s == NS - 1)
        def _():
            pltpu.async_copy(acc.at[pl.ds(15 * CP0, CP1)],
                             out_hbm.at[c, pl.ds(15 * CP0, CP1)], sem).wait()

    return agg


_agg64 = _make_agg(64)
_agg32 = _make_agg(32)


def _mm_body(x_ref, w_ref, o_ref):
    o_ref[...] = jnp.dot(x_ref[...], w_ref[...],
                         preferred_element_type=jnp.float32)


def _mm_xw(x, W):
    k, f = W.shape
    return pl.pallas_call(
        _mm_body,
        grid=(5,),
        in_specs=[pl.BlockSpec((2000, k), lambda i: (i, 0)),
                  pl.BlockSpec((k, f), lambda i: (0, 0))],
        out_specs=pl.BlockSpec((2000, f), lambda i: (i, 0)),
        out_shape=jax.ShapeDtypeStruct((N, f), jnp.float32),
    )(x, W)


def _dinv_body(p_ref, o_ref):
    s = jnp.sum(p_ref[...], axis=0, keepdims=True) + 1.0
    o_ref[...] = lax.rsqrt(s)


def _dinv_row(parts):
    return pl.pallas_call(
        _dinv_body,
        in_specs=[pl.BlockSpec((NW, NPAD), lambda: (0, 0))],
        out_specs=pl.BlockSpec((1, NPAD), lambda: (0, 0)),
        out_shape=jax.ShapeDtypeStruct((1, NPAD), jnp.float32),
    )(parts)


def _scale_body(a_ref, d_ref, o_ref):
    o_ref[...] = a_ref[...] * d_ref[...]


def _scale_rows(a, dcol):
    f = a.shape[1]
    return pl.pallas_call(
        _scale_body,
        grid=(5,),
        in_specs=[pl.BlockSpec((2000, f), lambda i: (i, 0)),
                  pl.BlockSpec((2000, 1), lambda i: (i, 0))],
        out_specs=pl.BlockSpec((2000, f), lambda i: (i, 0)),
        out_shape=jax.ShapeDtypeStruct((N, f), jnp.float32),
    )(a, dcol)


def _mid_body(p_ref, xs_ref, d_ref, b1_ref, w2_ref, o_ref):
    e = p_ref[0] + p_ref[1] + xs_ref[...]
    h = jnp.maximum(d_ref[...] * e + b1_ref[...], 0.0)
    o_ref[...] = d_ref[...] * jnp.dot(h, w2_ref[...],
                                      preferred_element_type=jnp.float32)


def _mid(p, xs1, dcol, b1, W2):
    return pl.pallas_call(
        _mid_body,
        grid=(5,),
        in_specs=[pl.BlockSpec((NC, 2000, 64), lambda i: (0, i, 0)),
                  pl.BlockSpec((2000, 64), lambda i: (i, 0)),
                  pl.BlockSpec((2000, 1), lambda i: (i, 0)),
                  pl.BlockSpec((1, 64), lambda i: (0, 0)),
                  pl.BlockSpec((64, 32), lambda i: (0, 0))],
        out_specs=pl.BlockSpec((2000, 32), lambda i: (i, 0)),
        out_shape=jax.ShapeDtypeStruct((N, 32), jnp.float32),
    )(p, xs1, dcol, b1, W2)


def _dec_body(q_ref, hs_ref, d_ref, b2_ref, wf1_ref, bf1_ref, wf2_ref,
              bf2_ref, o_ref):
    z = d_ref[...] * (q_ref[0] + q_ref[1] + hs_ref[...]) + b2_ref[...]
    dd = jnp.maximum(jnp.dot(z, wf1_ref[...],
                             preferred_element_type=jnp.float32)
                     + bf1_ref[...], 0.0)
    o_ref[...] = jnp.maximum(jnp.dot(dd, wf2_ref[...],
                                     preferred_element_type=jnp.float32)
                             + bf2_ref[...], 0.0)


def _decode(q, hs, dcol, b2, Wf1, bf1, Wf2, bf2):
    return pl.pallas_call(
        _dec_body,
        grid=(5,),
        in_specs=[pl.BlockSpec((NC, 2000, 32), lambda i: (0, i, 0)),
                  pl.BlockSpec((2000, 32), lambda i: (i, 0)),
                  pl.BlockSpec((2000, 1), lambda i: (i, 0)),
                  pl.BlockSpec((1, 32), lambda i: (0, 0)),
                  pl.BlockSpec((32, 64), lambda i: (0, 0)),
                  pl.BlockSpec((1, 64), lambda i: (0, 0)),
                  pl.BlockSpec((64, 128), lambda i: (0, 0)),
                  pl.BlockSpec((1, 128), lambda i: (0, 0))],
        out_specs=pl.BlockSpec((2000, 128), lambda i: (i, 0)),
        out_shape=jax.ShapeDtypeStruct((N, 128), jnp.float32),
    )(q, hs, dcol, b2, Wf1, bf1, Wf2, bf2)


def kernel(x, edge_index, W1, b1, W2, b2, Wf1, bf1, Wf2, bf2):
    ei = edge_index.astype(jnp.int32)
    src3 = ei[0].reshape(NW, NBLK, BLK)
    dst3 = ei[1].reshape(NW, NBLK, BLK)
    dst_deg = ei[1].reshape(NW, EPW // L, L)

    deg_parts = _deg_kernel(dst_deg)
    xw1 = _mm_xw(x, W1)

    dinv = _dinv_row(deg_parts)
    dcol = dinv[0, :N].reshape(N, 1)

    xs1 = _scale_rows(xw1, dcol)
    p = _agg64(xs1, src3, dst3)
    hs = _mid(p, xs1, dcol, b1.reshape(1, 64), W2)
    q = _agg32(hs, src3, dst3)
    return _decode(q, hs, dcol, b2.reshape(1, 32), Wf1, bf1.reshape(1, 64),
                   Wf2, bf2.reshape(1, 128))


# speedup vs baseline: 41.3220x; 1.4344x over previous
import dataclasses
import functools

import jax
import jax.numpy as jnp
from jax import lax
from jax.experimental import pallas as pl
from jax.experimental.pallas import tpu as pltpu
from jax.experimental.pallas import tpu_sc as plsc

N = 10000
E = 320000
NC, NS, L = 2, 16, 16
NW = NC * NS
EPW = E // NW
BLK = 100
NBLK = EPW // BLK
NPAD = 10240
ZR = 128
CP0 = 624
CP1 = N - 15 * CP0

_mesh = plsc.VectorSubcoreMesh(core_axis_name="c", subcore_axis_name="s")

_sc_params = pltpu.CompilerParams()
for _f, _v in (("needs_layout_passes", False), ("use_tc_tiling_on_sc", False)):
    if _f in pltpu.CompilerParams.__dataclass_fields__:
        _sc_params = dataclasses.replace(_sc_params, **{_f: _v})


@functools.partial(
    pl.kernel, mesh=_mesh, compiler_params=_sc_params,
    out_type=jax.ShapeDtypeStruct((NW, NPAD), jnp.float32),
    scratch_types=[
        pltpu.VMEM((EPW // L, L), jnp.int32),
        pltpu.VMEM((NPAD,), jnp.float32),
        pltpu.SemaphoreType.DMA,
    ],
)
def _deg_kernel(dst_hbm, out_hbm, dst_v, hist, sem):
    wid = lax.axis_index("s") * NC + lax.axis_index("c")
    pltpu.async_copy(dst_hbm.at[wid], dst_v, sem).wait()

    zero16 = jnp.zeros((L,), jnp.float32)

    @pl.loop(0, NPAD // L)
    def _(i):
        hist[pl.ds(i * L, L)] = zero16

    ones16 = jnp.ones((L,), jnp.float32)

    @pl.loop(0, EPW // L)
    def _(j):
        plsc.addupdate_scatter(hist, [dst_v[j, :]], ones16)

    pltpu.async_copy(hist, out_hbm.at[wid], sem).wait()


def _make_agg(F):
    @functools.partial(
        pl.kernel, mesh=_mesh, compiler_params=_sc_params,
        out_type=jax.ShapeDtypeStruct((NC, N, F), jnp.float32),
        scratch_types=[
            pltpu.VMEM((NBLK, BLK), jnp.int32),
            pltpu.VMEM((NBLK, BLK), jnp.int32),
            pltpu.VMEM((BLK, F), jnp.float32),
            pltpu.VMEM((BLK, F), jnp.float32),
            pltpu.VMEM((ZR, F), jnp.float32),
            pltpu.VMEM_SHARED((NPAD, F), jnp.float32),
            pltpu.SemaphoreType.DMA,
            pltpu.SemaphoreType.DMA,
            pltpu.SemaphoreType.DMA,
        ],
    )
    def agg(table_hbm, src_hbm, dst_hbm, out_hbm, src_v, dst_v, bufa, bufb,
            zbuf, acc, sema, semb, semi):
        c = lax.axis_index("c")
        s = lax.axis_index("s")
        wid = s * NC + c
        cpi = pltpu.async_copy(src_hbm.at[wid], src_v, semi)
        pltpu.async_copy(dst_hbm.at[wid], dst_v, semi)

        zero16 = jnp.zeros((L,), jnp.float32)

        @pl.loop(0, ZR)
        def _(r):
            @pl.loop(0, F // L)
            def _(k):
                zbuf[r, pl.ds(k * L, L)] = zero16

        @pl.loop(0, NPAD // NS // ZR)
        def _(t):
            pltpu.sync_copy(zbuf, acc.at[pl.ds(s * (NPAD // NS) + t * ZR, ZR)])

        cpi.wait()
        pltpu.make_async_copy(dst_hbm.at[wid], dst_v, semi).wait()
        plsc.subcore_barrier()

        pltpu.async_copy(table_hbm.at[src_v.at[0]], bufa, sema)

        @pl.loop(0, NBLK // 2)
        def _(j):
            b = 2 * j
            pltpu.async_copy(table_hbm.at[src_v.at[b + 1]], bufb, semb)
            pltpu.make_async_copy(table_hbm.at[src_v.at[b]], bufa, sema).wait()
            pltpu.sync_copy(bufa, acc.at[dst_v.at[b]], add=True)

            @pl.when(j < NBLK // 2 - 1)
            def _():
                pltpu.async_copy(table_hbm.at[src_v.at[b + 2]], bufa, sema)

            pltpu.make_async_copy(table_hbm.at[src_v.at[b + 1]], bufb,
                                  semb).wait()
            pltpu.sync_copy(bufb, acc.at[dst_v.at[b + 1]], add=True)

        plsc.subcore_barrier()

        @pl.when(s < NS - 1)
        def _():
            pltpu.async_copy(acc.at[pl.ds(s * CP0, CP0)],
                             out_hbm.at[c, pl.ds(s * CP0, CP0)], sema).wait()

        @pl.when(s == NS - 1)
        def _():
            pltpu.async_copy(acc.at[pl.ds(15 * CP0, CP1)],
                             out_hbm.at[c, pl.ds(15 * CP0, CP1)], sema).wait()

    return agg


_agg64 = _make_agg(64)
_agg32 = _make_agg(32)


def _mm_body(x_ref, w_ref, o_ref):
    o_ref[...] = jnp.dot(x_ref[...], w_ref[...],
                         preferred_element_type=jnp.float32)


def _mm_xw(x, W):
    k, f = W.shape
    return pl.pallas_call(
        _mm_body,
        grid=(5,),
        in_specs=[pl.BlockSpec((2000, k), lambda i: (i, 0)),
                  pl.BlockSpec((k, f), lambda i: (0, 0))],
        out_specs=pl.BlockSpec((2000, f), lambda i: (i, 0)),
        out_shape=jax.ShapeDtypeStruct((N, f), jnp.float32),
    )(x, W)


def _dinv_body(p_ref, o_ref):
    s = jnp.sum(p_ref[...], axis=0, keepdims=True) + 1.0
    o_ref[...] = lax.rsqrt(s)


def _dinv_row(parts):
    return pl.pallas_call(
        _dinv_body,
        in_specs=[pl.BlockSpec((NW, NPAD), lambda: (0, 0))],
        out_specs=pl.BlockSpec((1, NPAD), lambda: (0, 0)),
        out_shape=jax.ShapeDtypeStruct((1, NPAD), jnp.float32),
    )(parts)


def _scale_body(a_ref, d_ref, o_ref):
    o_ref[...] = a_ref[...] * d_ref[...]


def _scale_rows(a, dcol):
    f = a.shape[1]
    return pl.pallas_call(
        _scale_body,
        grid=(5,),
        in_specs=[pl.BlockSpec((2000, f), lambda i: (i, 0)),
                  pl.BlockSpec((2000, 1), lambda i: (i, 0))],
        out_specs=pl.BlockSpec((2000, f), lambda i: (i, 0)),
        out_shape=jax.ShapeDtypeStruct((N, f), jnp.float32),
    )(a, dcol)


def _mid_body(p_ref, xs_ref, d_ref, b1_ref, w2_ref, o_ref):
    e = p_ref[0] + p_ref[1] + xs_ref[...]
    h = jnp.maximum(d_ref[...] * e + b1_ref[...], 0.0)
    o_ref[...] = d_ref[...] * jnp.dot(h, w2_ref[...],
                                      preferred_element_type=jnp.float32)


def _mid(p, xs1, dcol, b1, W2):
    return pl.pallas_call(
        _mid_body,
        grid=(5,),
        in_specs=[pl.BlockSpec((NC, 2000, 64), lambda i: (0, i, 0)),
                  pl.BlockSpec((2000, 64), lambda i: (i, 0)),
                  pl.BlockSpec((2000, 1), lambda i: (i, 0)),
                  pl.BlockSpec((1, 64), lambda i: (0, 0)),
                  pl.BlockSpec((64, 32), lambda i: (0, 0))],
        out_specs=pl.BlockSpec((2000, 32), lambda i: (i, 0)),
        out_shape=jax.ShapeDtypeStruct((N, 32), jnp.float32),
    )(p, xs1, dcol, b1, W2)


def _dec_body(q_ref, hs_ref, d_ref, b2_ref, wf1_ref, bf1_ref, wf2_ref,
              bf2_ref, o_ref):
    z = d_ref[...] * (q_ref[0] + q_ref[1] + hs_ref[...]) + b2_ref[...]
    dd = jnp.maximum(jnp.dot(z, wf1_ref[...],
                             preferred_element_type=jnp.float32)
                     + bf1_ref[...], 0.0)
    o_ref[...] = jnp.maximum(jnp.dot(dd, wf2_ref[...],
                                     preferred_element_type=jnp.float32)
                             + bf2_ref[...], 0.0)


def _decode(q, hs, dcol, b2, Wf1, bf1, Wf2, bf2):
    return pl.pallas_call(
        _dec_body,
        grid=(5,),
        in_specs=[pl.BlockSpec((NC, 2000, 32), lambda i: (0, i, 0)),
                  pl.BlockSpec((2000, 32), lambda i: (i, 0)),
                  pl.BlockSpec((2000, 1), lambda i: (i, 0)),
                  pl.BlockSpec((1, 32), lambda i: (0, 0)),
                  pl.BlockSpec((32, 64), lambda i: (0, 0)),
                  pl.BlockSpec((1, 64), lambda i: (0, 0)),
                  pl.BlockSpec((64, 128), lambda i: (0, 0)),
                  pl.BlockSpec((1, 128), lambda i: (0, 0))],
        out_specs=pl.BlockSpec((2000, 128), lambda i: (i, 0)),
        out_shape=jax.ShapeDtypeStruct((N, 128), jnp.float32),
    )(q, hs, dcol, b2, Wf1, bf1, Wf2, bf2)


def kernel(x, edge_index, W1, b1, W2, b2, Wf1, bf1, Wf2, bf2):
    ei = edge_index.astype(jnp.int32)
    src3 = ei[0].reshape(NW, NBLK, BLK)
    dst3 = ei[1].reshape(NW, NBLK, BLK)
    dst_deg = ei[1].reshape(NW, EPW // L, L)

    deg_parts = _deg_kernel(dst_deg)
    xw1 = _mm_xw(x, W1)

    dinv = _dinv_row(deg_parts)
    dcol = dinv[0, :N].reshape(N, 1)

    xs1 = _scale_rows(xw1, dcol)
    p = _agg64(xs1, src3, dst3)
    hs = _mid(p, xs1, dcol, b1.reshape(1, 64), W2)
    q = _agg32(hs, src3, dst3)
    return _decode(q, hs, dcol, b2.reshape(1, 32), Wf1, bf1.reshape(1, 64),
                   Wf2, bf2.reshape(1, 128))
